# baseline (speedup 1.0000x reference)
import functools

import jax
import jax.numpy as jnp
import numpy as np
from jax import lax
from jax.experimental import pallas as pl
from jax.experimental.pallas import tpu as pltpu
from jax.experimental.pallas import tpu_sc as plsc

N_NODES = 10000
D = 128
D2 = 256
N_GRAPHS = 64
EPS_MSG = 1e-7
BN_EPS = 1e-5
BLK = 1000

NC, NS, L = 2, 16, 16
CHUNK = 112
CPT = 186
E_PAD = NS * CPT * CHUNK
NROWS = 10240
RPT = NROWS // NS
ZCH = 80
EA_W = 512


def _tc_edge_feat_body(ea_ref, w_ref, b_ref, out_ref):
    out_ref[...] = jnp.dot(ea_ref[...], w_ref[...],
                           preferred_element_type=jnp.float32) + b_ref[...]


def _tc_edge_feat(ea2d, tWe, tbe):
    eblk = 8192
    grid = (E_PAD // eblk,)
    return pl.pallas_call(
        _tc_edge_feat_body,
        grid=grid,
        in_specs=[pl.BlockSpec((eblk, 4), lambda i: (i, 0)),
                  pl.BlockSpec((4, D), lambda i: (0, 0)),
                  pl.BlockSpec((1, D), lambda i: (0, 0))],
        out_specs=pl.BlockSpec((eblk, D), lambda i: (i, 0)),
        out_shape=jax.ShapeDtypeStruct((E_PAD, D), jnp.float32),
    )(ea2d, tWe, tbe)


def _tc_layer_body(s1_ref, s2_ref, tinv_ref, g_ref, xres_ref, w1_ref, b1_ref,
                   w2_ref, b2_ref, nw_ref, nb_ref, x_out_ref, g_out_ref, *,
                   has_res):
    out = s2_ref[...] * tinv_ref[...] / (s1_ref[...] + 1e-16) + g_ref[...]
    h = jnp.dot(out, w1_ref[...], preferred_element_type=jnp.float32) + b1_ref[...]
    h = jnp.maximum(h, 0.0)
    h = jnp.dot(h, w2_ref[...], preferred_element_type=jnp.float32) + b2_ref[...]
    if has_res:
        h = h + xres_ref[...]
    x_out_ref[...] = h
    gnext = h * nw_ref[...] + nb_ref[...]
    g_out_ref[...] = jnp.where(gnext >= 0.0, gnext, 0.01 * gnext)


def _tc_layer(s1, s2, tinv, g, xres, w1, b1, w2, b2, nw, nb, has_res):
    grid = (N_NODES // BLK,)
    row = pl.BlockSpec((BLK, D), lambda i: (i, 0))
    full = lambda shape: pl.BlockSpec(shape, lambda i: (0,) * len(shape))
    return pl.pallas_call(
        functools.partial(_tc_layer_body, has_res=has_res),
        grid=grid,
        in_specs=[row, row, full((1, D)), row, row, full((D, D2)),
                  full((1, D2)), full((D2, D)), full((1, D)), full((1, D)),
                  full((1, D))],
        out_specs=[row, row],
        out_shape=[jax.ShapeDtypeStruct((N_NODES, D), jnp.float32)] * 2,
    )(s1, s2, tinv, g, xres, w1, b1, w2, b2, nw, nb)


def _tc_final_body(s1_ref, s2_ref, tinv_ref, g_ref, xres_ref, w1_ref, b1_ref,
                   w2_ref, b2_ref, nw_ref, nb_ref, batch_ref, out_ref, acc_ref,
                   cnt_ref):
    i = pl.program_id(0)

    @pl.when(i == 0)
    def _():
        acc_ref[...] = jnp.zeros_like(acc_ref)
        cnt_ref[...] = jnp.zeros_like(cnt_ref)

    out = s2_ref[...] * tinv_ref[...] / (s1_ref[...] + 1e-16) + g_ref[...]
    h = jnp.dot(out, w1_ref[...], preferred_element_type=jnp.float32) + b1_ref[...]
    h = jnp.maximum(h, 0.0)
    h = jnp.dot(h, w2_ref[...], preferred_element_type=jnp.float32) + b2_ref[...]
    h = h + xres_ref[...]
    y = h * nw_ref[...] + nb_ref[...]
    y = jnp.where(y >= 0.0, y, 0.01 * y)
    gids = lax.broadcasted_iota(jnp.int32, (N_GRAPHS, BLK), 0)
    onehot = (batch_ref[0] == gids).astype(jnp.float32)
    acc_ref[...] += jnp.dot(onehot, y, preferred_element_type=jnp.float32)
    cnt_ref[...] += jnp.sum(onehot, axis=1, keepdims=True)

    @pl.when(i == pl.num_programs(0) - 1)
    def _():
        out_ref[...] = acc_ref[...] / jnp.maximum(cnt_ref[...], 1.0)


def _tc_final(s1, s2, tinv, g, xres, w1, b1, w2, b2, nw, nb, batch3d):
    grid = (N_NODES // BLK,)
    row = pl.BlockSpec((BLK, D), lambda i: (i, 0))
    full = lambda shape: pl.BlockSpec(shape, lambda i: (0,) * len(shape))
    return pl.pallas_call(
        _tc_final_body,
        grid=grid,
        in_specs=[row, row, full((1, D)), row, row, full((D, D2)),
                  full((1, D2)), full((D2, D)), full((1, D)), full((1, D)),
                  full((1, D)), pl.BlockSpec((1, 1, BLK), lambda i: (i, 0, 0))],
        out_specs=full((N_GRAPHS, D)),
        out_shape=jax.ShapeDtypeStruct((N_GRAPHS, D), jnp.float32),
        scratch_shapes=[pltpu.VMEM((N_GRAPHS, D), jnp.float32),
                        pltpu.VMEM((N_GRAPHS, 1), jnp.float32)],
    )(s1, s2, tinv, g, xres, w1, b1, w2, b2, nw, nb, batch3d)


def _sc_edge_body(g_hbm, src_hbm, dst_hbm, ea_hbm, pp_hbm, out_hbm,
                  src_v, src_b, src_c, dst_v, dst_b, dst_c, ea_v,
                  rows_v, rows_b, rows_c, pp_v,
                  sem, sem_b, sem_c, sem_s0, sem_s1, sem_s2, acc_sh):
    c = lax.axis_index("c")
    t = lax.axis_index("s")
    pltpu.sync_copy(pp_hbm, pp_v)

    zero = jnp.zeros((L,), jnp.float32)

    def zb(j, _):
        for s in range(8):
            rows_v[j, pl.ds(16 * s, 16)] = zero
        return 0
    lax.fori_loop(0, CHUNK, zb, 0)

    def zs(k, _):
        pltpu.sync_copy(rows_v.at[pl.ds(0, ZCH)],
                        acc_sh.at[pl.ds(t * RPT + k * ZCH, ZCH)])
        return 0
    lax.fori_loop(0, RPT // ZCH, zs, 0)
    plsc.subcore_barrier()

    we = [[pp_v[k, pl.ds(16 * s, 16)] for s in range(8)] for k in range(4)]
    bev = [pp_v[4, pl.ds(16 * s, 16)] for s in range(8)]
    pv = pp_v[5, pl.ds(0, 16)]
    t_sc = pv[0]
    teps = pv[1]
    cf = jnp.full((L,), c, jnp.int32).astype(jnp.float32)
    s0v = 1.0 - cf
    s1v = cf

    def start_gather(k, src_ref, rows_ref, semx):
        pltpu.sync_copy(src_hbm.at[t * CPT + k], src_ref)
        pltpu.async_copy(g_hbm.at[src_ref], rows_ref, semx)

    def wait_gather(src_ref, rows_ref, semx):
        pltpu.make_async_copy(g_hbm.at[src_ref], rows_ref, semx).wait()

    def start_scatter(rows_ref, dst_ref, semx):
        pltpu.async_copy(rows_ref, acc_sh.at[dst_ref], semx, add=True)

    def wait_scatter(rows_ref, dst_ref, semx):
        pltpu.make_async_copy(rows_ref, acc_sh.at[dst_ref], semx).wait()

    def compute_chunk(k, rows_ref, dst_ref):
        ck = t * CPT + k
        pltpu.sync_copy(dst_hbm.at[ck], dst_ref)
        pltpu.sync_copy(ea_hbm.at[ck], ea_v.at[pl.ds(0, EA_W)])

        def edge_body(j, _):
            av = ea_v[pl.ds(4 * j, 16)]
            a0 = av[0]
            a1 = av[1]
            a2 = av[2]
            a3 = av[3]
            for s in range(8):
                sl = pl.ds(16 * s, 16)
                ev = a0 * we[0][s] + a1 * we[1][s] + a2 * we[2][s] \
                    + a3 * we[3][s] + bev[s]
                u = jnp.maximum(t_sc * rows_ref[j, sl] + ev, 0.0) + teps
                w = jnp.exp(u)
                rows_ref[j, sl] = w * (s0v + s1v * u)
            return 0
        lax.fori_loop(0, CHUNK, edge_body, 0)

    start_gather(0, src_v, rows_v, sem)
    start_gather(1, src_b, rows_b, sem_b)
    start_gather(2, src_c, rows_c, sem_c)
    ni = CPT // 3

    def ring_body(i, _):
        a = 3 * i
        wait_gather(src_v, rows_v, sem)
        compute_chunk(a, rows_v, dst_v)
        start_scatter(rows_v, dst_v, sem_s0)

        @pl.when(i > 0)
        def _():
            wait_scatter(rows_c, dst_c, sem_s2)
            start_gather(a + 2, src_c, rows_c, sem_c)

        wait_gather(src_b, rows_b, sem_b)
        compute_chunk(a + 1, rows_b, dst_b)
        start_scatter(rows_b, dst_b, sem_s1)
        wait_scatter(rows_v, dst_v, sem_s0)

        @pl.when(i < ni - 1)
        def _():
            start_gather(a + 3, src_v, rows_v, sem)

        wait_gather(src_c, rows_c, sem_c)
        compute_chunk(a + 2, rows_c, dst_c)
        start_scatter(rows_c, dst_c, sem_s2)
        wait_scatter(rows_b, dst_b, sem_s1)

        @pl.when(i < ni - 1)
        def _():
            start_gather(a + 4, src_b, rows_b, sem_b)

        return 0
    lax.fori_loop(0, ni, ring_body, 0)
    wait_scatter(rows_c, dst_c, sem_s2)
    plsc.subcore_barrier()

    def wb(k, _):
        r = t * RPT + k * ZCH
        pltpu.sync_copy(acc_sh.at[pl.ds(r, ZCH)], rows_v.at[pl.ds(0, ZCH)])
        pltpu.sync_copy(rows_v.at[pl.ds(0, ZCH)],
                        out_hbm.at[pl.ds(c * NROWS + r, ZCH)])
        return 0
    lax.fori_loop(0, RPT // ZCH, wb, 0)


def _sc_edge(g, src2, dst2, ea2, pp):
    f32 = jnp.float32
    mesh = plsc.VectorSubcoreMesh(core_axis_name="c", subcore_axis_name="s",
                                  num_cores=NC, num_subcores=NS)
    kern = pl.kernel(
        _sc_edge_body,
        out_type=jax.ShapeDtypeStruct((2 * NROWS, D), f32),
        mesh=mesh,
        scratch_types=[
            pltpu.VMEM((CHUNK,), jnp.int32),
            pltpu.VMEM((CHUNK,), jnp.int32),
            pltpu.VMEM((CHUNK,), jnp.int32),
            pltpu.VMEM((CHUNK,), jnp.int32),
            pltpu.VMEM((CHUNK,), jnp.int32),
            pltpu.VMEM((CHUNK,), jnp.int32),
            pltpu.VMEM((EA_W + 16,), f32),
            pltpu.VMEM((CHUNK, D), f32),
            pltpu.VMEM((CHUNK, D), f32),
            pltpu.VMEM((CHUNK, D), f32),
            pltpu.VMEM((6, D), f32),
            pltpu.SemaphoreType.DMA,
            pltpu.SemaphoreType.DMA,
            pltpu.SemaphoreType.DMA,
            pltpu.SemaphoreType.DMA,
            pltpu.SemaphoreType.DMA,
            pltpu.SemaphoreType.DMA,
            pltpu.VMEM_SHARED((NROWS, D), f32),
        ],
    )
    res = kern(g, src2, dst2, ea2, pp)
    return res[:N_NODES], res[NROWS:NROWS + N_NODES]


def _edge_phase(g, src2, dst2, ea2, We, be, t):
    scal = jnp.concatenate([jnp.full((1, 1), t, jnp.float32),
                            jnp.full((1, 1), t * EPS_MSG, jnp.float32),
                            jnp.zeros((1, D - 2), jnp.float32)], axis=1)
    pp = jnp.concatenate([t * We, (t * be)[None, :], scal], axis=0)
    return _sc_edge(g, src2, dst2, ea2, pp)


def kernel(x, edge_index, edge_attr, batch, clinical, params):
    del clinical
    src, dst = edge_index[0], edge_index[1]
    n_edges = src.shape[0]
    npad = E_PAD - n_edges
    src2 = jnp.concatenate([src.astype(jnp.int32),
                            jnp.zeros((npad,), jnp.int32)]).reshape(NS * CPT, CHUNK)
    dst2 = jnp.concatenate([dst.astype(jnp.int32),
                            jnp.full((npad,), N_NODES, jnp.int32)]).reshape(NS * CPT, CHUNK)
    ea2 = jnp.concatenate([edge_attr.astype(jnp.float32),
                           jnp.zeros((npad, 4), jnp.float32)]).reshape(
                               NS * CPT, 4 * CHUNK)
    ea2 = jnp.concatenate(
        [ea2, jnp.zeros((NS * CPT, EA_W - 4 * CHUNK), jnp.float32)], axis=1)
    bns = 1.0 / np.sqrt(1.0 + BN_EPS)

    def folded(i):
        p = params[f"conv{i}"]
        s = p["bn1_w"] * bns
        w1 = p["W1"] * s[None, :]
        b1 = (p["b1"] * s + p["bn1_b"])[None, :]
        w2 = p["W2"]
        b2 = p["b2"][None, :]
        return w1, b1, w2, b2

    def norm(name):
        nm = params[name]
        return (nm["w"] * bns)[None, :], nm["b"][None, :]

    nw1, nb1 = norm("norm1")
    nw2, nb2 = norm("norm2")
    nw0, nb0 = norm("norm0")
    batch3d = batch.astype(jnp.int32).reshape(N_NODES // BLK, 1, BLK)

    tinvs = [jnp.full((1, D), 1.0, jnp.float32) / params[f"conv{i}"]["t"]
             for i in range(3)]

    p0 = params["conv0"]
    s1, s2 = _edge_phase(x, src2, dst2, ea2, p0["We"], p0["be"], p0["t"])
    x1, g1 = _tc_layer(s1, s2, tinvs[0], x, x, *folded(0), nw1, nb1,
                       has_res=False)
    p1 = params["conv1"]
    s1, s2 = _edge_phase(g1, src2, dst2, ea2, p1["We"], p1["be"], p1["t"])
    x2, g2 = _tc_layer(s1, s2, tinvs[1], g1, x1, *folded(1), nw2, nb2,
                       has_res=True)
    p2 = params["conv2"]
    s1, s2 = _edge_phase(g2, src2, dst2, ea2, p2["We"], p2["be"], p2["t"])
    return _tc_final(s1, s2, tinvs[2], g2, x2, *folded(2), nw0, nb0, batch3d)


# speedup vs baseline: 1.3602x; 1.3602x over previous
import functools

import jax
import jax.numpy as jnp
import numpy as np
from jax import lax
from jax.experimental import pallas as pl
from jax.experimental.pallas import tpu as pltpu
from jax.experimental.pallas import tpu_sc as plsc

N_NODES = 10000
D = 128
D2 = 256
N_GRAPHS = 64
EPS_MSG = 1e-7
BN_EPS = 1e-5
BLK = 1000

NC, NS, L = 2, 16, 16
CHUNK = 80
CPT = 250
E_PAD = NS * CPT * CHUNK
NROWS = 10240
RPT = NROWS // NS
ZCH = 80
EA_W = 384


def _tc_edge_feat_body(ea_ref, w_ref, b_ref, out_ref):
    out_ref[...] = jnp.dot(ea_ref[...], w_ref[...],
                           preferred_element_type=jnp.float32) + b_ref[...]


def _tc_edge_feat(ea2d, tWe, tbe):
    eblk = 8192
    grid = (E_PAD // eblk,)
    return pl.pallas_call(
        _tc_edge_feat_body,
        grid=grid,
        in_specs=[pl.BlockSpec((eblk, 4), lambda i: (i, 0)),
                  pl.BlockSpec((4, D), lambda i: (0, 0)),
                  pl.BlockSpec((1, D), lambda i: (0, 0))],
        out_specs=pl.BlockSpec((eblk, D), lambda i: (i, 0)),
        out_shape=jax.ShapeDtypeStruct((E_PAD, D), jnp.float32),
    )(ea2d, tWe, tbe)


def _tc_layer_body(s1_ref, s2_ref, tinv_ref, g_ref, xres_ref, w1_ref, b1_ref,
                   w2_ref, b2_ref, nw_ref, nb_ref, x_out_ref, g_out_ref, *,
                   has_res):
    out = s2_ref[...] * tinv_ref[...] / (s1_ref[...] + 1e-16) + g_ref[...]
    h = jnp.dot(out, w1_ref[...], preferred_element_type=jnp.float32) + b1_ref[...]
    h = jnp.maximum(h, 0.0)
    h = jnp.dot(h, w2_ref[...], preferred_element_type=jnp.float32) + b2_ref[...]
    if has_res:
        h = h + xres_ref[...]
    x_out_ref[...] = h
    gnext = h * nw_ref[...] + nb_ref[...]
    g_out_ref[...] = jnp.where(gnext >= 0.0, gnext, 0.01 * gnext)


def _tc_layer(s1, s2, tinv, g, xres, w1, b1, w2, b2, nw, nb, has_res):
    grid = (N_NODES // BLK,)
    row = pl.BlockSpec((BLK, D), lambda i: (i, 0))
    full = lambda shape: pl.BlockSpec(shape, lambda i: (0,) * len(shape))
    return pl.pallas_call(
        functools.partial(_tc_layer_body, has_res=has_res),
        grid=grid,
        in_specs=[row, row, full((1, D)), row, row, full((D, D2)),
                  full((1, D2)), full((D2, D)), full((1, D)), full((1, D)),
                  full((1, D))],
        out_specs=[row, row],
        out_shape=[jax.ShapeDtypeStruct((N_NODES, D), jnp.float32)] * 2,
    )(s1, s2, tinv, g, xres, w1, b1, w2, b2, nw, nb)


def _tc_final_body(s1_ref, s2_ref, tinv_ref, g_ref, xres_ref, w1_ref, b1_ref,
                   w2_ref, b2_ref, nw_ref, nb_ref, batch_ref, out_ref, acc_ref,
                   cnt_ref):
    i = pl.program_id(0)

    @pl.when(i == 0)
    def _():
        acc_ref[...] = jnp.zeros_like(acc_ref)
        cnt_ref[...] = jnp.zeros_like(cnt_ref)

    out = s2_ref[...] * tinv_ref[...] / (s1_ref[...] + 1e-16) + g_ref[...]
    h = jnp.dot(out, w1_ref[...], preferred_element_type=jnp.float32) + b1_ref[...]
    h = jnp.maximum(h, 0.0)
    h = jnp.dot(h, w2_ref[...], preferred_element_type=jnp.float32) + b2_ref[...]
    h = h + xres_ref[...]
    y = h * nw_ref[...] + nb_ref[...]
    y = jnp.where(y >= 0.0, y, 0.01 * y)
    gids = lax.broadcasted_iota(jnp.int32, (N_GRAPHS, BLK), 0)
    onehot = (batch_ref[0] == gids).astype(jnp.float32)
    acc_ref[...] += jnp.dot(onehot, y, preferred_element_type=jnp.float32)
    cnt_ref[...] += jnp.sum(onehot, axis=1, keepdims=True)

    @pl.when(i == pl.num_programs(0) - 1)
    def _():
        out_ref[...] = acc_ref[...] / jnp.maximum(cnt_ref[...], 1.0)


def _tc_final(s1, s2, tinv, g, xres, w1, b1, w2, b2, nw, nb, batch3d):
    grid = (N_NODES // BLK,)
    row = pl.BlockSpec((BLK, D), lambda i: (i, 0))
    full = lambda shape: pl.BlockSpec(shape, lambda i: (0,) * len(shape))
    return pl.pallas_call(
        _tc_final_body,
        grid=grid,
        in_specs=[row, row, full((1, D)), row, row, full((D, D2)),
                  full((1, D2)), full((D2, D)), full((1, D)), full((1, D)),
                  full((1, D)), pl.BlockSpec((1, 1, BLK), lambda i: (i, 0, 0))],
        out_specs=full((N_GRAPHS, D)),
        out_shape=jax.ShapeDtypeStruct((N_GRAPHS, D), jnp.float32),
        scratch_shapes=[pltpu.VMEM((N_GRAPHS, D), jnp.float32),
                        pltpu.VMEM((N_GRAPHS, 1), jnp.float32)],
    )(s1, s2, tinv, g, xres, w1, b1, w2, b2, nw, nb, batch3d)


def _sc_edge_body(g_hbm, src_hbm, dst_hbm, ea_hbm, pp_hbm, out_hbm,
                  src_a, src_b, dst_a, dst_b, ea_v, rows_a, rows_b,
                  buf_a, buf_b, pp_v, sem_ga, sem_gb, sem_sa, sem_sb, acc_sh):
    c = lax.axis_index("c")
    t = lax.axis_index("s")
    pltpu.sync_copy(pp_hbm, pp_v)

    zero = jnp.zeros((L,), jnp.float32)

    def zb(j, _):
        for s in range(8):
            buf_a[j, pl.ds(16 * s, 16)] = zero
            buf_b[j, pl.ds(16 * s, 16)] = zero
        return 0
    lax.fori_loop(0, CHUNK, zb, 0)

    def zs(k, _):
        pltpu.sync_copy(buf_a.at[pl.ds(0, ZCH)],
                        acc_sh.at[pl.ds(t * RPT + k * ZCH, ZCH)])
        return 0
    lax.fori_loop(0, RPT // ZCH, zs, 0)
    plsc.subcore_barrier()

    we = [[pp_v[k, pl.ds(16 * s, 16)] for s in range(8)] for k in range(4)]
    bev = [pp_v[4, pl.ds(16 * s, 16)] for s in range(8)]
    pv = pp_v[5, pl.ds(0, 16)]
    t_sc = pv[0]
    teps = pv[1]
    cf = jnp.full((L,), c, jnp.int32).astype(jnp.float32)
    s0v = 1.0 - cf
    s1v = cf

    def start_gather(k, src_ref, rows_ref, semx):
        pltpu.sync_copy(src_hbm.at[t * CPT + k], src_ref)
        pltpu.async_copy(g_hbm.at[src_ref], rows_ref, semx)

    def wait_gather(src_ref, rows_ref, semx):
        pltpu.make_async_copy(g_hbm.at[src_ref], rows_ref, semx).wait()

    def start_scatter(buf_ref, dst_ref, semx):
        pltpu.async_copy(buf_ref, acc_sh.at[dst_ref], semx, add=True)

    def wait_scatter(buf_ref, dst_ref, semx):
        pltpu.make_async_copy(buf_ref, acc_sh.at[dst_ref], semx).wait()

    def compute_chunk(k, rows_ref, buf_ref, dst_ref):
        ck = t * CPT + k
        pltpu.sync_copy(dst_hbm.at[ck], dst_ref)
        pltpu.sync_copy(ea_hbm.at[ck], ea_v.at[pl.ds(0, EA_W)])

        def edge_body(j, _):
            av = ea_v[pl.ds(4 * j, 16)]
            a0 = av[0]
            a1 = av[1]
            a2 = av[2]
            a3 = av[3]
            for s in range(8):
                sl = pl.ds(16 * s, 16)
                ev = a0 * we[0][s] + a1 * we[1][s] + a2 * we[2][s] \
                    + a3 * we[3][s] + bev[s]
                u = jnp.maximum(t_sc * rows_ref[j, sl] + ev, 0.0) + teps
                w = jnp.exp(u)
                buf_ref[j, sl] = w * (s0v + s1v * u)
            return 0
        lax.fori_loop(0, CHUNK, edge_body, 0)

    def arm(j, _):
        trash = jnp.full((L,), N_NODES, jnp.int32)
        dst_a[pl.ds(j * 16, 16)] = trash
        dst_b[pl.ds(j * 16, 16)] = trash
        return 0
    lax.fori_loop(0, CHUNK // 16, arm, 0)
    start_scatter(buf_a, dst_a, sem_sa)
    start_scatter(buf_b, dst_b, sem_sb)
    start_gather(0, src_a, rows_a, sem_ga)

    def pair_body(i, _):
        k0 = 2 * i
        wait_gather(src_a, rows_a, sem_ga)
        start_gather(k0 + 1, src_b, rows_b, sem_gb)
        wait_scatter(buf_a, dst_a, sem_sa)
        compute_chunk(k0, rows_a, buf_a, dst_a)
        start_scatter(buf_a, dst_a, sem_sa)
        wait_gather(src_b, rows_b, sem_gb)

        @pl.when(i < CPT // 2 - 1)
        def _():
            start_gather(k0 + 2, src_a, rows_a, sem_ga)

        wait_scatter(buf_b, dst_b, sem_sb)
        compute_chunk(k0 + 1, rows_b, buf_b, dst_b)
        start_scatter(buf_b, dst_b, sem_sb)
        return 0
    lax.fori_loop(0, CPT // 2, pair_body, 0)
    wait_scatter(buf_a, dst_a, sem_sa)
    wait_scatter(buf_b, dst_b, sem_sb)
    plsc.subcore_barrier()

    def wb(k, _):
        r = t * RPT + k * ZCH
        pltpu.sync_copy(acc_sh.at[pl.ds(r, ZCH)], buf_a.at[pl.ds(0, ZCH)])
        pltpu.sync_copy(buf_a.at[pl.ds(0, ZCH)],
                        out_hbm.at[pl.ds(c * NROWS + r, ZCH)])
        return 0
    lax.fori_loop(0, RPT // ZCH, wb, 0)


def _sc_edge(g, src2, dst2, ea2, pp):
    f32 = jnp.float32
    mesh = plsc.VectorSubcoreMesh(core_axis_name="c", subcore_axis_name="s",
                                  num_cores=NC, num_subcores=NS)
    kern = pl.kernel(
        _sc_edge_body,
        out_type=jax.ShapeDtypeStruct((2 * NROWS, D), f32),
        mesh=mesh,
        scratch_types=[
            pltpu.VMEM((CHUNK,), jnp.int32),
            pltpu.VMEM((CHUNK,), jnp.int32),
            pltpu.VMEM((CHUNK,), jnp.int32),
            pltpu.VMEM((CHUNK,), jnp.int32),
            pltpu.VMEM((EA_W + 16,), f32),
            pltpu.VMEM((CHUNK, D), f32),
            pltpu.VMEM((CHUNK, D), f32),
            pltpu.VMEM((CHUNK, D), f32),
            pltpu.VMEM((CHUNK, D), f32),
            pltpu.VMEM((6, D), f32),
            pltpu.SemaphoreType.DMA,
            pltpu.SemaphoreType.DMA,
            pltpu.SemaphoreType.DMA,
            pltpu.SemaphoreType.DMA,
            pltpu.VMEM_SHARED((NROWS, D), f32),
        ],
    )
    res = kern(g, src2, dst2, ea2, pp)
    return res[:N_NODES], res[NROWS:NROWS + N_NODES]


def _edge_phase(g, src2, dst2, ea2, We, be, t):
    scal = jnp.concatenate([jnp.full((1, 1), t, jnp.float32),
                            jnp.full((1, 1), t * EPS_MSG, jnp.float32),
                            jnp.zeros((1, D - 2), jnp.float32)], axis=1)
    pp = jnp.concatenate([t * We, (t * be)[None, :], scal], axis=0)
    return _sc_edge(g, src2, dst2, ea2, pp)


def kernel(x, edge_index, edge_attr, batch, clinical, params):
    del clinical
    src, dst = edge_index[0], edge_index[1]
    n_edges = src.shape[0]
    npad = E_PAD - n_edges
    src2 = jnp.concatenate([src.astype(jnp.int32),
                            jnp.zeros((npad,), jnp.int32)]).reshape(NS * CPT, CHUNK)
    dst2 = jnp.concatenate([dst.astype(jnp.int32),
                            jnp.full((npad,), N_NODES, jnp.int32)]).reshape(NS * CPT, CHUNK)
    ea2 = jnp.concatenate([edge_attr.astype(jnp.float32),
                           jnp.zeros((npad, 4), jnp.float32)]).reshape(
                               NS * CPT, 4 * CHUNK)
    ea2 = jnp.concatenate(
        [ea2, jnp.zeros((NS * CPT, EA_W - 4 * CHUNK), jnp.float32)], axis=1)
    bns = 1.0 / np.sqrt(1.0 + BN_EPS)

    def folded(i):
        p = params[f"conv{i}"]
        s = p["bn1_w"] * bns
        w1 = p["W1"] * s[None, :]
        b1 = (p["b1"] * s + p["bn1_b"])[None, :]
        w2 = p["W2"]
        b2 = p["b2"][None, :]
        return w1, b1, w2, b2

    def norm(name):
        nm = params[name]
        return (nm["w"] * bns)[None, :], nm["b"][None, :]

    nw1, nb1 = norm("norm1")
    nw2, nb2 = norm("norm2")
    nw0, nb0 = norm("norm0")
    batch3d = batch.astype(jnp.int32).reshape(N_NODES // BLK, 1, BLK)

    tinvs = [jnp.full((1, D), 1.0, jnp.float32) / params[f"conv{i}"]["t"]
             for i in range(3)]

    p0 = params["conv0"]
    s1, s2 = _edge_phase(x, src2, dst2, ea2, p0["We"], p0["be"], p0["t"])
    x1, g1 = _tc_layer(s1, s2, tinvs[0], x, x, *folded(0), nw1, nb1,
                       has_res=False)
    p1 = params["conv1"]
    s1, s2 = _edge_phase(g1, src2, dst2, ea2, p1["We"], p1["be"], p1["t"])
    x2, g2 = _tc_layer(s1, s2, tinvs[1], g1, x1, *folded(1), nw2, nb2,
                       has_res=True)
    p2 = params["conv2"]
    s1, s2 = _edge_phase(g2, src2, dst2, ea2, p2["We"], p2["be"], p2["t"])
    return _tc_final(s1, s2, tinvs[2], g2, x2, *folded(2), nw0, nb0, batch3d)


# speedup vs baseline: 1.3854x; 1.0185x over previous
import functools

import jax
import jax.numpy as jnp
import numpy as np
from jax import lax
from jax.experimental import pallas as pl
from jax.experimental.pallas import tpu as pltpu
from jax.experimental.pallas import tpu_sc as plsc

N_NODES = 10000
D = 128
D2 = 256
N_GRAPHS = 64
EPS_MSG = 1e-7
BN_EPS = 1e-5
BLK = 1000

NC, NS, L = 2, 16, 16
CHUNK = 80
CPT = 250
E_PAD = NS * CPT * CHUNK
NROWS = 10240
RPT = NROWS // NS
ZCH = 80
EA_W = 384


def _tc_edge_feat_body(ea_ref, w_ref, b_ref, out_ref):
    out_ref[...] = jnp.dot(ea_ref[...], w_ref[...],
                           preferred_element_type=jnp.float32) + b_ref[...]


def _tc_edge_feat(ea2d, tWe, tbe):
    eblk = 8192
    grid = (E_PAD // eblk,)
    return pl.pallas_call(
        _tc_edge_feat_body,
        grid=grid,
        in_specs=[pl.BlockSpec((eblk, 4), lambda i: (i, 0)),
                  pl.BlockSpec((4, D), lambda i: (0, 0)),
                  pl.BlockSpec((1, D), lambda i: (0, 0))],
        out_specs=pl.BlockSpec((eblk, D), lambda i: (i, 0)),
        out_shape=jax.ShapeDtypeStruct((E_PAD, D), jnp.float32),
    )(ea2d, tWe, tbe)


def _tc_layer_body(s1_ref, s2_ref, tinv_ref, g_ref, xres_ref, w1_ref, b1_ref,
                   w2_ref, b2_ref, nw_ref, nb_ref, x_out_ref, g_out_ref, *,
                   has_res):
    out = s2_ref[...] * tinv_ref[...] / (s1_ref[...] + 1e-16) + g_ref[...]
    h = jnp.dot(out, w1_ref[...], preferred_element_type=jnp.float32) + b1_ref[...]
    h = jnp.maximum(h, 0.0)
    h = jnp.dot(h, w2_ref[...], preferred_element_type=jnp.float32) + b2_ref[...]
    if has_res:
        h = h + xres_ref[...]
    x_out_ref[...] = h
    gnext = h * nw_ref[...] + nb_ref[...]
    g_out_ref[...] = jnp.where(gnext >= 0.0, gnext, 0.01 * gnext)


def _tc_layer(s1, s2, tinv, g, xres, w1, b1, w2, b2, nw, nb, has_res):
    grid = (N_NODES // BLK,)
    row = pl.BlockSpec((BLK, D), lambda i: (i, 0))
    full = lambda shape: pl.BlockSpec(shape, lambda i: (0,) * len(shape))
    return pl.pallas_call(
        functools.partial(_tc_layer_body, has_res=has_res),
        grid=grid,
        in_specs=[row, row, full((1, D)), row, row, full((D, D2)),
                  full((1, D2)), full((D2, D)), full((1, D)), full((1, D)),
                  full((1, D))],
        out_specs=[row, row],
        out_shape=[jax.ShapeDtypeStruct((N_NODES, D), jnp.float32)] * 2,
    )(s1, s2, tinv, g, xres, w1, b1, w2, b2, nw, nb)


def _tc_final_body(s1_ref, s2_ref, tinv_ref, g_ref, xres_ref, w1_ref, b1_ref,
                   w2_ref, b2_ref, nw_ref, nb_ref, batch_ref, out_ref, acc_ref,
                   cnt_ref):
    i = pl.program_id(0)

    @pl.when(i == 0)
    def _():
        acc_ref[...] = jnp.zeros_like(acc_ref)
        cnt_ref[...] = jnp.zeros_like(cnt_ref)

    out = s2_ref[...] * tinv_ref[...] / (s1_ref[...] + 1e-16) + g_ref[...]
    h = jnp.dot(out, w1_ref[...], preferred_element_type=jnp.float32) + b1_ref[...]
    h = jnp.maximum(h, 0.0)
    h = jnp.dot(h, w2_ref[...], preferred_element_type=jnp.float32) + b2_ref[...]
    h = h + xres_ref[...]
    y = h * nw_ref[...] + nb_ref[...]
    y = jnp.where(y >= 0.0, y, 0.01 * y)
    gids = lax.broadcasted_iota(jnp.int32, (N_GRAPHS, BLK), 0)
    onehot = (batch_ref[0] == gids).astype(jnp.float32)
    acc_ref[...] += jnp.dot(onehot, y, preferred_element_type=jnp.float32)
    cnt_ref[...] += jnp.sum(onehot, axis=1, keepdims=True)

    @pl.when(i == pl.num_programs(0) - 1)
    def _():
        out_ref[...] = acc_ref[...] / jnp.maximum(cnt_ref[...], 1.0)


def _tc_final(s1, s2, tinv, g, xres, w1, b1, w2, b2, nw, nb, batch3d):
    grid = (N_NODES // BLK,)
    row = pl.BlockSpec((BLK, D), lambda i: (i, 0))
    full = lambda shape: pl.BlockSpec(shape, lambda i: (0,) * len(shape))
    return pl.pallas_call(
        _tc_final_body,
        grid=grid,
        in_specs=[row, row, full((1, D)), row, row, full((D, D2)),
                  full((1, D2)), full((D2, D)), full((1, D)), full((1, D)),
                  full((1, D)), pl.BlockSpec((1, 1, BLK), lambda i: (i, 0, 0))],
        out_specs=full((N_GRAPHS, D)),
        out_shape=jax.ShapeDtypeStruct((N_GRAPHS, D), jnp.float32),
        scratch_shapes=[pltpu.VMEM((N_GRAPHS, D), jnp.float32),
                        pltpu.VMEM((N_GRAPHS, 1), jnp.float32)],
    )(s1, s2, tinv, g, xres, w1, b1, w2, b2, nw, nb, batch3d)


def _sc_edge_body(g_hbm, src_hbm, dst_hbm, ea_hbm, pp_hbm, out_hbm,
                  src_a, src_b, dst_a, dst_b, ea_v, rows_a, rows_b,
                  buf_a, buf_b, pp_v, sem_ga, sem_gb, sem_sa, sem_sb, acc_sh):
    c = lax.axis_index("c")
    t = lax.axis_index("s")
    pltpu.sync_copy(pp_hbm, pp_v)

    zero = jnp.zeros((L,), jnp.float32)

    def zb(j, _):
        for s in range(8):
            buf_a[j, pl.ds(16 * s, 16)] = zero
            buf_b[j, pl.ds(16 * s, 16)] = zero
        return 0
    lax.fori_loop(0, CHUNK, zb, 0)

    def zs(k, _):
        pltpu.sync_copy(buf_a.at[pl.ds(0, ZCH)],
                        acc_sh.at[pl.ds(t * RPT + k * ZCH, ZCH)])
        return 0
    lax.fori_loop(0, RPT // ZCH, zs, 0)
    plsc.subcore_barrier()

    we = [[pp_v[k, pl.ds(16 * s, 16)] for s in range(8)] for k in range(4)]
    bev = [pp_v[4, pl.ds(16 * s, 16)] for s in range(8)]
    pv = pp_v[5, pl.ds(0, 16)]
    t_sc = pv[0]
    teps = pv[1]
    cf = jnp.full((L,), c, jnp.int32).astype(jnp.float32)
    s0v = 1.0 - cf
    s1v = cf

    def start_gather(k, src_ref, rows_ref, semx):
        pltpu.sync_copy(src_hbm.at[t * CPT + k], src_ref)
        pltpu.async_copy(g_hbm.at[src_ref], rows_ref, semx)

    def wait_gather(src_ref, rows_ref, semx):
        pltpu.make_async_copy(g_hbm.at[src_ref], rows_ref, semx).wait()

    def start_scatter(buf_ref, dst_ref, semx):
        pltpu.async_copy(buf_ref, acc_sh.at[dst_ref], semx, add=True)

    def wait_scatter(buf_ref, dst_ref, semx):
        pltpu.make_async_copy(buf_ref, acc_sh.at[dst_ref], semx).wait()

    def compute_chunk(k, rows_ref, buf_ref, dst_ref):
        ck = t * CPT + k
        pltpu.sync_copy(dst_hbm.at[ck], dst_ref)
        pltpu.sync_copy(ea_hbm.at[ck], ea_v.at[pl.ds(0, EA_W)])

        def edge_body(j):
            av = ea_v[pl.ds(4 * j, 16)]
            a0 = av[0]
            a1 = av[1]
            a2 = av[2]
            a3 = av[3]
            for s in range(8):
                sl = pl.ds(16 * s, 16)
                ev = a0 * we[0][s] + a1 * we[1][s] + a2 * we[2][s] \
                    + a3 * we[3][s] + bev[s]
                u = jnp.maximum(t_sc * rows_ref[j, sl] + ev, 0.0) + teps
                w = jnp.exp(u)
                buf_ref[j, sl] = w * (s0v + s1v * u)
        plsc.parallel_loop(0, CHUNK, 1, unroll=2)(edge_body)

    def arm(j, _):
        trash = jnp.full((L,), N_NODES, jnp.int32)
        dst_a[pl.ds(j * 16, 16)] = trash
        dst_b[pl.ds(j * 16, 16)] = trash
        return 0
    lax.fori_loop(0, CHUNK // 16, arm, 0)
    start_scatter(buf_a, dst_a, sem_sa)
    start_scatter(buf_b, dst_b, sem_sb)
    start_gather(0, src_a, rows_a, sem_ga)

    def pair_body(i, _):
        k0 = 2 * i
        wait_gather(src_a, rows_a, sem_ga)
        start_gather(k0 + 1, src_b, rows_b, sem_gb)
        wait_scatter(buf_a, dst_a, sem_sa)
        compute_chunk(k0, rows_a, buf_a, dst_a)
        start_scatter(buf_a, dst_a, sem_sa)
        wait_gather(src_b, rows_b, sem_gb)

        @pl.when(i < CPT // 2 - 1)
        def _():
            start_gather(k0 + 2, src_a, rows_a, sem_ga)

        wait_scatter(buf_b, dst_b, sem_sb)
        compute_chunk(k0 + 1, rows_b, buf_b, dst_b)
        start_scatter(buf_b, dst_b, sem_sb)
        return 0
    lax.fori_loop(0, CPT // 2, pair_body, 0)
    wait_scatter(buf_a, dst_a, sem_sa)
    wait_scatter(buf_b, dst_b, sem_sb)
    plsc.subcore_barrier()

    def wb(k, _):
        r = t * RPT + k * ZCH
        pltpu.sync_copy(acc_sh.at[pl.ds(r, ZCH)], buf_a.at[pl.ds(0, ZCH)])
        pltpu.sync_copy(buf_a.at[pl.ds(0, ZCH)],
                        out_hbm.at[pl.ds(c * NROWS + r, ZCH)])
        return 0
    lax.fori_loop(0, RPT // ZCH, wb, 0)


def _sc_edge(g, src2, dst2, ea2, pp):
    f32 = jnp.float32
    mesh = plsc.VectorSubcoreMesh(core_axis_name="c", subcore_axis_name="s",
                                  num_cores=NC, num_subcores=NS)
    kern = pl.kernel(
        _sc_edge_body,
        out_type=jax.ShapeDtypeStruct((2 * NROWS, D), f32),
        mesh=mesh,
        scratch_types=[
            pltpu.VMEM((CHUNK,), jnp.int32),
            pltpu.VMEM((CHUNK,), jnp.int32),
            pltpu.VMEM((CHUNK,), jnp.int32),
            pltpu.VMEM((CHUNK,), jnp.int32),
            pltpu.VMEM((EA_W + 16,), f32),
            pltpu.VMEM((CHUNK, D), f32),
            pltpu.VMEM((CHUNK, D), f32),
            pltpu.VMEM((CHUNK, D), f32),
            pltpu.VMEM((CHUNK, D), f32),
            pltpu.VMEM((6, D), f32),
            pltpu.SemaphoreType.DMA,
            pltpu.SemaphoreType.DMA,
            pltpu.SemaphoreType.DMA,
            pltpu.SemaphoreType.DMA,
            pltpu.VMEM_SHARED((NROWS, D), f32),
        ],
    )
    res = kern(g, src2, dst2, ea2, pp)
    return res[:N_NODES], res[NROWS:NROWS + N_NODES]


def _edge_phase(g, src2, dst2, ea2, We, be, t):
    scal = jnp.concatenate([jnp.full((1, 1), t, jnp.float32),
                            jnp.full((1, 1), t * EPS_MSG, jnp.float32),
                            jnp.zeros((1, D - 2), jnp.float32)], axis=1)
    pp = jnp.concatenate([t * We, (t * be)[None, :], scal], axis=0)
    return _sc_edge(g, src2, dst2, ea2, pp)


def kernel(x, edge_index, edge_attr, batch, clinical, params):
    del clinical
    src, dst = edge_index[0], edge_index[1]
    n_edges = src.shape[0]
    npad = E_PAD - n_edges
    src2 = jnp.concatenate([src.astype(jnp.int32),
                            jnp.zeros((npad,), jnp.int32)]).reshape(NS * CPT, CHUNK)
    dst2 = jnp.concatenate([dst.astype(jnp.int32),
                            jnp.full((npad,), N_NODES, jnp.int32)]).reshape(NS * CPT, CHUNK)
    ea2 = jnp.concatenate([edge_attr.astype(jnp.float32),
                           jnp.zeros((npad, 4), jnp.float32)]).reshape(
                               NS * CPT, 4 * CHUNK)
    ea2 = jnp.concatenate(
        [ea2, jnp.zeros((NS * CPT, EA_W - 4 * CHUNK), jnp.float32)], axis=1)
    bns = 1.0 / np.sqrt(1.0 + BN_EPS)

    def folded(i):
        p = params[f"conv{i}"]
        s = p["bn1_w"] * bns
        w1 = p["W1"] * s[None, :]
        b1 = (p["b1"] * s + p["bn1_b"])[None, :]
        w2 = p["W2"]
        b2 = p["b2"][None, :]
        return w1, b1, w2, b2

    def norm(name):
        nm = params[name]
        return (nm["w"] * bns)[None, :], nm["b"][None, :]

    nw1, nb1 = norm("norm1")
    nw2, nb2 = norm("norm2")
    nw0, nb0 = norm("norm0")
    batch3d = batch.astype(jnp.int32).reshape(N_NODES // BLK, 1, BLK)

    tinvs = [jnp.full((1, D), 1.0, jnp.float32) / params[f"conv{i}"]["t"]
             for i in range(3)]

    p0 = params["conv0"]
    s1, s2 = _edge_phase(x, src2, dst2, ea2, p0["We"], p0["be"], p0["t"])
    x1, g1 = _tc_layer(s1, s2, tinvs[0], x, x, *folded(0), nw1, nb1,
                       has_res=False)
    p1 = params["conv1"]
    s1, s2 = _edge_phase(g1, src2, dst2, ea2, p1["We"], p1["be"], p1["t"])
    x2, g2 = _tc_layer(s1, s2, tinvs[1], g1, x1, *folded(1), nw2, nb2,
                       has_res=True)
    p2 = params["conv2"]
    s1, s2 = _edge_phase(g2, src2, dst2, ea2, p2["We"], p2["be"], p2["t"])
    return _tc_final(s1, s2, tinvs[2], g2, x2, *folded(2), nw0, nb0, batch3d)


# speedup vs baseline: 1.8466x; 1.3329x over previous
import functools

import jax
import jax.numpy as jnp
import numpy as np
from jax import lax
from jax.experimental import pallas as pl
from jax.experimental.pallas import tpu as pltpu
from jax.experimental.pallas import tpu_sc as plsc

N_NODES = 10000
D = 128
D2 = 256
N_GRAPHS = 64
EPS_MSG = 1e-7
BN_EPS = 1e-5
BLK = 1000

NC, NS, L = 2, 16, 16
CHUNK = 64
CPT = 314
E_PAD = NS * CPT * CHUNK
NROWS = 10112
RPT = NROWS // NS
ZCH = 56
ZN = 11
ZTAIL = RPT - ZN * ZCH
EBLK = 10048


def _tc_edge_feat_body(ea_ref, w_ref, b_ref, out_ref):
    out_ref[...] = jnp.dot(ea_ref[...], w_ref[...],
                           preferred_element_type=jnp.float32) + b_ref[...]


def _tc_edge_feat(ea2d, tWe, tbe):
    grid = (E_PAD // EBLK,)
    return pl.pallas_call(
        _tc_edge_feat_body,
        grid=grid,
        in_specs=[pl.BlockSpec((EBLK, 4), lambda i: (i, 0)),
                  pl.BlockSpec((4, D), lambda i: (0, 0)),
                  pl.BlockSpec((1, D), lambda i: (0, 0))],
        out_specs=pl.BlockSpec((EBLK, D), lambda i: (i, 0)),
        out_shape=jax.ShapeDtypeStruct((E_PAD, D), jnp.float32),
    )(ea2d, tWe, tbe)


def _tc_layer_body(s1_ref, s2_ref, tinv_ref, g_ref, xres_ref, w1_ref, b1_ref,
                   w2_ref, b2_ref, nw_ref, nb_ref, x_out_ref, g_out_ref, *,
                   has_res):
    out = s2_ref[...] * tinv_ref[...] / (s1_ref[...] + 1e-16) + g_ref[...]
    h = jnp.dot(out, w1_ref[...], preferred_element_type=jnp.float32) + b1_ref[...]
    h = jnp.maximum(h, 0.0)
    h = jnp.dot(h, w2_ref[...], preferred_element_type=jnp.float32) + b2_ref[...]
    if has_res:
        h = h + xres_ref[...]
    x_out_ref[...] = h
    gnext = h * nw_ref[...] + nb_ref[...]
    g_out_ref[...] = jnp.where(gnext >= 0.0, gnext, 0.01 * gnext)


def _tc_layer(s1, s2, tinv, g, xres, w1, b1, w2, b2, nw, nb, has_res):
    grid = (N_NODES // BLK,)
    row = pl.BlockSpec((BLK, D), lambda i: (i, 0))
    full = lambda shape: pl.BlockSpec(shape, lambda i: (0,) * len(shape))
    return pl.pallas_call(
        functools.partial(_tc_layer_body, has_res=has_res),
        grid=grid,
        in_specs=[row, row, full((1, D)), row, row, full((D, D2)),
                  full((1, D2)), full((D2, D)), full((1, D)), full((1, D)),
                  full((1, D))],
        out_specs=[row, row],
        out_shape=[jax.ShapeDtypeStruct((N_NODES, D), jnp.float32)] * 2,
    )(s1, s2, tinv, g, xres, w1, b1, w2, b2, nw, nb)


def _tc_final_body(s1_ref, s2_ref, tinv_ref, g_ref, xres_ref, w1_ref, b1_ref,
                   w2_ref, b2_ref, nw_ref, nb_ref, batch_ref, out_ref, acc_ref,
                   cnt_ref):
    i = pl.program_id(0)

    @pl.when(i == 0)
    def _():
        acc_ref[...] = jnp.zeros_like(acc_ref)
        cnt_ref[...] = jnp.zeros_like(cnt_ref)

    out = s2_ref[...] * tinv_ref[...] / (s1_ref[...] + 1e-16) + g_ref[...]
    h = jnp.dot(out, w1_ref[...], preferred_element_type=jnp.float32) + b1_ref[...]
    h = jnp.maximum(h, 0.0)
    h = jnp.dot(h, w2_ref[...], preferred_element_type=jnp.float32) + b2_ref[...]
    h = h + xres_ref[...]
    y = h * nw_ref[...] + nb_ref[...]
    y = jnp.where(y >= 0.0, y, 0.01 * y)
    gids = lax.broadcasted_iota(jnp.int32, (N_GRAPHS, BLK), 0)
    onehot = (batch_ref[0] == gids).astype(jnp.float32)
    acc_ref[...] += jnp.dot(onehot, y, preferred_element_type=jnp.float32)
    cnt_ref[...] += jnp.sum(onehot, axis=1, keepdims=True)

    @pl.when(i == pl.num_programs(0) - 1)
    def _():
        out_ref[...] = acc_ref[...] / jnp.maximum(cnt_ref[...], 1.0)


def _tc_final(s1, s2, tinv, g, xres, w1, b1, w2, b2, nw, nb, batch3d):
    grid = (N_NODES // BLK,)
    row = pl.BlockSpec((BLK, D), lambda i: (i, 0))
    full = lambda shape: pl.BlockSpec(shape, lambda i: (0,) * len(shape))
    return pl.pallas_call(
        _tc_final_body,
        grid=grid,
        in_specs=[row, row, full((1, D)), row, row, full((D, D2)),
                  full((1, D2)), full((D2, D)), full((1, D)), full((1, D)),
                  full((1, D)), pl.BlockSpec((1, 1, BLK), lambda i: (i, 0, 0))],
        out_specs=full((N_GRAPHS, D)),
        out_shape=jax.ShapeDtypeStruct((N_GRAPHS, D), jnp.float32),
        scratch_shapes=[pltpu.VMEM((N_GRAPHS, D), jnp.float32),
                        pltpu.VMEM((N_GRAPHS, 1), jnp.float32)],
    )(s1, s2, tinv, g, xres, w1, b1, w2, b2, nw, nb, batch3d)


def _sc_edge_body(g_hbm, src_hbm, dst_hbm, e_hbm, pp_hbm, out_hbm,
                  src_a, src_b, dst_a, dst_b, e_a, e_b, rows_a, rows_b,
                  buf_a, buf_b, pp_v, sem_ga, sem_gb, sem_ea, sem_eb,
                  sem_sa, sem_sb, acc_sh):
    c = lax.axis_index("c")
    t = lax.axis_index("s")
    pltpu.sync_copy(pp_hbm, pp_v)

    zero = jnp.zeros((L,), jnp.float32)

    def zb(j, _):
        for s in range(8):
            buf_a[j, pl.ds(16 * s, 16)] = zero
            buf_b[j, pl.ds(16 * s, 16)] = zero
        return 0
    lax.fori_loop(0, CHUNK, zb, 0)

    def zs(k, _):
        pltpu.sync_copy(buf_a.at[pl.ds(0, ZCH)],
                        acc_sh.at[pl.ds(t * RPT + k * ZCH, ZCH)])
        return 0
    lax.fori_loop(0, ZN, zs, 0)
    pltpu.sync_copy(buf_a.at[pl.ds(0, ZTAIL)],
                    acc_sh.at[pl.ds(t * RPT + ZN * ZCH, ZTAIL)])
    plsc.subcore_barrier()

    pv = pp_v[0, pl.ds(0, 16)]
    t_sc = pv[0]
    teps = pv[1]
    cf = jnp.full((L,), c, jnp.int32).astype(jnp.float32)
    s0v = 1.0 - cf
    s1v = cf

    def start_side(k, src_ref, rows_ref, e_ref, semg, seme):
        ck = t * CPT + k
        pltpu.sync_copy(src_hbm.at[ck], src_ref)
        pltpu.async_copy(g_hbm.at[src_ref], rows_ref, semg)
        pltpu.async_copy(e_hbm.at[pl.ds(ck * CHUNK, CHUNK)], e_ref, seme)

    def wait_side(src_ref, rows_ref, e_ref, semg, seme):
        pltpu.make_async_copy(g_hbm.at[src_ref], rows_ref, semg).wait()
        pltpu.make_async_copy(e_hbm.at[pl.ds(0, CHUNK)], e_ref, seme).wait()

    def start_scatter(buf_ref, dst_ref, semx):
        pltpu.async_copy(buf_ref, acc_sh.at[dst_ref], semx, add=True)

    def wait_scatter(buf_ref, dst_ref, semx):
        pltpu.make_async_copy(buf_ref, acc_sh.at[dst_ref], semx).wait()

    def compute_chunk(k, rows_ref, e_ref, buf_ref, dst_ref):
        pltpu.sync_copy(dst_hbm.at[t * CPT + k], dst_ref)

        def edge_body(j):
            for s in range(8):
                sl = pl.ds(16 * s, 16)
                u = jnp.maximum(t_sc * rows_ref[j, sl] + e_ref[j, sl],
                                0.0) + teps
                w = jnp.exp(u)
                buf_ref[j, sl] = w * (s0v + s1v * u)
        plsc.parallel_loop(0, CHUNK, 1, unroll=2)(edge_body)

    def arm(j, _):
        trash = jnp.full((L,), N_NODES, jnp.int32)
        dst_a[pl.ds(j * 16, 16)] = trash
        dst_b[pl.ds(j * 16, 16)] = trash
        return 0
    lax.fori_loop(0, CHUNK // 16, arm, 0)
    start_scatter(buf_a, dst_a, sem_sa)
    start_scatter(buf_b, dst_b, sem_sb)
    start_side(0, src_a, rows_a, e_a, sem_ga, sem_ea)

    def pair_body(i, _):
        k0 = 2 * i
        wait_side(src_a, rows_a, e_a, sem_ga, sem_ea)
        start_side(k0 + 1, src_b, rows_b, e_b, sem_gb, sem_eb)
        wait_scatter(buf_a, dst_a, sem_sa)
        compute_chunk(k0, rows_a, e_a, buf_a, dst_a)
        start_scatter(buf_a, dst_a, sem_sa)
        wait_side(src_b, rows_b, e_b, sem_gb, sem_eb)

        @pl.when(i < CPT // 2 - 1)
        def _():
            start_side(k0 + 2, src_a, rows_a, e_a, sem_ga, sem_ea)

        wait_scatter(buf_b, dst_b, sem_sb)
        compute_chunk(k0 + 1, rows_b, e_b, buf_b, dst_b)
        start_scatter(buf_b, dst_b, sem_sb)
        return 0
    lax.fori_loop(0, CPT // 2, pair_body, 0)
    wait_scatter(buf_a, dst_a, sem_sa)
    wait_scatter(buf_b, dst_b, sem_sb)
    plsc.subcore_barrier()

    def wb(k, _):
        r = t * RPT + k * ZCH
        pltpu.sync_copy(acc_sh.at[pl.ds(r, ZCH)], buf_a.at[pl.ds(0, ZCH)])
        pltpu.sync_copy(buf_a.at[pl.ds(0, ZCH)],
                        out_hbm.at[pl.ds(c * NROWS + r, ZCH)])
        return 0
    lax.fori_loop(0, ZN, wb, 0)
    rt = t * RPT + ZN * ZCH
    pltpu.sync_copy(acc_sh.at[pl.ds(rt, ZTAIL)], buf_a.at[pl.ds(0, ZTAIL)])
    pltpu.sync_copy(buf_a.at[pl.ds(0, ZTAIL)],
                    out_hbm.at[pl.ds(c * NROWS + rt, ZTAIL)])


def _sc_edge(g, src2, dst2, e2, pp):
    f32 = jnp.float32
    mesh = plsc.VectorSubcoreMesh(core_axis_name="c", subcore_axis_name="s",
                                  num_cores=NC, num_subcores=NS)
    kern = pl.kernel(
        _sc_edge_body,
        out_type=jax.ShapeDtypeStruct((2 * NROWS, D), f32),
        mesh=mesh,
        scratch_types=[
            pltpu.VMEM((CHUNK,), jnp.int32),
            pltpu.VMEM((CHUNK,), jnp.int32),
            pltpu.VMEM((CHUNK,), jnp.int32),
            pltpu.VMEM((CHUNK,), jnp.int32),
            pltpu.VMEM((CHUNK, D), f32),
            pltpu.VMEM((CHUNK, D), f32),
            pltpu.VMEM((CHUNK, D), f32),
            pltpu.VMEM((CHUNK, D), f32),
            pltpu.VMEM((CHUNK, D), f32),
            pltpu.VMEM((CHUNK, D), f32),
            pltpu.VMEM((1, D), f32),
            pltpu.SemaphoreType.DMA,
            pltpu.SemaphoreType.DMA,
            pltpu.SemaphoreType.DMA,
            pltpu.SemaphoreType.DMA,
            pltpu.SemaphoreType.DMA,
            pltpu.SemaphoreType.DMA,
            pltpu.VMEM_SHARED((NROWS, D), f32),
        ],
    )
    res = kern(g, src2, dst2, e2, pp)
    return res[:N_NODES], res[NROWS:NROWS + N_NODES]


def _edge_phase(g, src2, dst2, e2, t):
    pp = jnp.concatenate([jnp.full((1, 1), t, jnp.float32),
                          jnp.full((1, 1), t * EPS_MSG, jnp.float32),
                          jnp.zeros((1, D - 2), jnp.float32)], axis=1)
    return _sc_edge(g, src2, dst2, e2, pp)


def kernel(x, edge_index, edge_attr, batch, clinical, params):
    del clinical
    src, dst = edge_index[0], edge_index[1]
    n_edges = src.shape[0]
    npad = E_PAD - n_edges
    src2 = jnp.concatenate([src.astype(jnp.int32),
                            jnp.zeros((npad,), jnp.int32)]).reshape(NS * CPT, CHUNK)
    dst2 = jnp.concatenate([dst.astype(jnp.int32),
                            jnp.full((npad,), N_NODES, jnp.int32)]).reshape(NS * CPT, CHUNK)
    ea2d = jnp.concatenate([edge_attr.astype(jnp.float32),
                            jnp.zeros((npad, 4), jnp.float32)])
    bns = 1.0 / np.sqrt(1.0 + BN_EPS)

    def folded(i):
        p = params[f"conv{i}"]
        s = p["bn1_w"] * bns
        w1 = p["W1"] * s[None, :]
        b1 = (p["b1"] * s + p["bn1_b"])[None, :]
        w2 = p["W2"]
        b2 = p["b2"][None, :]
        return w1, b1, w2, b2

    def norm(name):
        nm = params[name]
        return (nm["w"] * bns)[None, :], nm["b"][None, :]

    nw1, nb1 = norm("norm1")
    nw2, nb2 = norm("norm2")
    nw0, nb0 = norm("norm0")
    batch3d = batch.astype(jnp.int32).reshape(N_NODES // BLK, 1, BLK)

    tinvs = [jnp.full((1, D), 1.0, jnp.float32) / params[f"conv{i}"]["t"]
             for i in range(3)]
    eprimes = []
    for i in range(3):
        p = params[f"conv{i}"]
        eprimes.append(_tc_edge_feat(ea2d, p["t"] * p["We"],
                                     (p["t"] * p["be"])[None, :]))

    s1, s2 = _edge_phase(x, src2, dst2, eprimes[0], params["conv0"]["t"])
    x1, g1 = _tc_layer(s1, s2, tinvs[0], x, x, *folded(0), nw1, nb1,
                       has_res=False)
    s1, s2 = _edge_phase(g1, src2, dst2, eprimes[1], params["conv1"]["t"])
    x2, g2 = _tc_layer(s1, s2, tinvs[1], g1, x1, *folded(1), nw2, nb2,
                       has_res=True)
    s1, s2 = _edge_phase(g2, src2, dst2, eprimes[2], params["conv2"]["t"])
    return _tc_final(s1, s2, tinvs[2], g2, x2, *folded(2), nw0, nb0, batch3d)


# speedup vs baseline: 1.8492x; 1.0014x over previous
import functools

import jax
import jax.numpy as jnp
import numpy as np
from jax import lax
from jax.experimental import pallas as pl
from jax.experimental.pallas import tpu as pltpu
from jax.experimental.pallas import tpu_sc as plsc

N_NODES = 10000
D = 128
D2 = 256
N_GRAPHS = 64
EPS_MSG = 1e-7
BN_EPS = 1e-5
BLK = 1000

NC, NS, L = 2, 16, 16
CHUNK = 64
CPT = 314
E_PAD = NS * CPT * CHUNK
NROWS = 10112
RPT = NROWS // NS
ZCH = 56
ZN = 11
ZTAIL = RPT - ZN * ZCH
EBLK = 10048


def _tc_edge_feat_body(ea_ref, w_ref, b_ref, out_ref):
    out_ref[...] = jnp.dot(ea_ref[...], w_ref[...],
                           preferred_element_type=jnp.float32) + b_ref[...]


def _tc_edge_feat(ea2d, tWe, tbe):
    grid = (E_PAD // EBLK,)
    return pl.pallas_call(
        _tc_edge_feat_body,
        grid=grid,
        in_specs=[pl.BlockSpec((EBLK, 4), lambda i: (i, 0)),
                  pl.BlockSpec((4, D), lambda i: (0, 0)),
                  pl.BlockSpec((1, D), lambda i: (0, 0))],
        out_specs=pl.BlockSpec((EBLK, D), lambda i: (i, 0)),
        out_shape=jax.ShapeDtypeStruct((E_PAD, D), jnp.float32),
    )(ea2d, tWe, tbe)


def _tc_layer_body(s1_ref, s2_ref, tinv_ref, g_ref, xres_ref, w1_ref, b1_ref,
                   w2_ref, b2_ref, nw_ref, nb_ref, x_out_ref, g_out_ref, *,
                   has_res):
    out = s2_ref[...] * tinv_ref[...] / (s1_ref[...] + 1e-16) + g_ref[...]
    h = jnp.dot(out, w1_ref[...], preferred_element_type=jnp.float32) + b1_ref[...]
    h = jnp.maximum(h, 0.0)
    h = jnp.dot(h, w2_ref[...], preferred_element_type=jnp.float32) + b2_ref[...]
    if has_res:
        h = h + xres_ref[...]
    x_out_ref[...] = h
    gnext = h * nw_ref[...] + nb_ref[...]
    g_out_ref[...] = jnp.where(gnext >= 0.0, gnext, 0.01 * gnext)


def _tc_layer(s1, s2, tinv, g, xres, w1, b1, w2, b2, nw, nb, has_res):
    grid = (N_NODES // BLK,)
    row = pl.BlockSpec((BLK, D), lambda i: (i, 0))
    full = lambda shape: pl.BlockSpec(shape, lambda i: (0,) * len(shape))
    return pl.pallas_call(
        functools.partial(_tc_layer_body, has_res=has_res),
        grid=grid,
        in_specs=[row, row, full((1, D)), row, row, full((D, D2)),
                  full((1, D2)), full((D2, D)), full((1, D)), full((1, D)),
                  full((1, D))],
        out_specs=[row, row],
        out_shape=[jax.ShapeDtypeStruct((N_NODES, D), jnp.float32)] * 2,
    )(s1, s2, tinv, g, xres, w1, b1, w2, b2, nw, nb)


def _tc_final_body(s1_ref, s2_ref, tinv_ref, g_ref, xres_ref, w1_ref, b1_ref,
                   w2_ref, b2_ref, nw_ref, nb_ref, batch_ref, out_ref, acc_ref,
                   cnt_ref):
    i = pl.program_id(0)

    @pl.when(i == 0)
    def _():
        acc_ref[...] = jnp.zeros_like(acc_ref)
        cnt_ref[...] = jnp.zeros_like(cnt_ref)

    out = s2_ref[...] * tinv_ref[...] / (s1_ref[...] + 1e-16) + g_ref[...]
    h = jnp.dot(out, w1_ref[...], preferred_element_type=jnp.float32) + b1_ref[...]
    h = jnp.maximum(h, 0.0)
    h = jnp.dot(h, w2_ref[...], preferred_element_type=jnp.float32) + b2_ref[...]
    h = h + xres_ref[...]
    y = h * nw_ref[...] + nb_ref[...]
    y = jnp.where(y >= 0.0, y, 0.01 * y)
    gids = lax.broadcasted_iota(jnp.int32, (N_GRAPHS, BLK), 0)
    onehot = (batch_ref[0] == gids).astype(jnp.float32)
    acc_ref[...] += jnp.dot(onehot, y, preferred_element_type=jnp.float32)
    cnt_ref[...] += jnp.sum(onehot, axis=1, keepdims=True)

    @pl.when(i == pl.num_programs(0) - 1)
    def _():
        out_ref[...] = acc_ref[...] / jnp.maximum(cnt_ref[...], 1.0)


def _tc_final(s1, s2, tinv, g, xres, w1, b1, w2, b2, nw, nb, batch3d):
    grid = (N_NODES // BLK,)
    row = pl.BlockSpec((BLK, D), lambda i: (i, 0))
    full = lambda shape: pl.BlockSpec(shape, lambda i: (0,) * len(shape))
    return pl.pallas_call(
        _tc_final_body,
        grid=grid,
        in_specs=[row, row, full((1, D)), row, row, full((D, D2)),
                  full((1, D2)), full((D2, D)), full((1, D)), full((1, D)),
                  full((1, D)), pl.BlockSpec((1, 1, BLK), lambda i: (i, 0, 0))],
        out_specs=full((N_GRAPHS, D)),
        out_shape=jax.ShapeDtypeStruct((N_GRAPHS, D), jnp.float32),
        scratch_shapes=[pltpu.VMEM((N_GRAPHS, D), jnp.float32),
                        pltpu.VMEM((N_GRAPHS, 1), jnp.float32)],
    )(s1, s2, tinv, g, xres, w1, b1, w2, b2, nw, nb, batch3d)


def _sc_edge_body(g_hbm, src_hbm, dst_hbm, e_hbm, pp_hbm, out_hbm,
                  src_a, src_b, dst_a, dst_b, e_a, e_b, rows_a, rows_b,
                  buf_a, buf_b, pp_v, sem_ga, sem_gb, sem_ea, sem_eb,
                  sem_sa, sem_sb, acc_sh):
    c = lax.axis_index("c")
    t = lax.axis_index("s")
    pltpu.sync_copy(pp_hbm, pp_v)

    zero = jnp.zeros((L,), jnp.float32)

    def zb(j, _):
        for s in range(8):
            buf_a[j, pl.ds(16 * s, 16)] = zero
            buf_b[j, pl.ds(16 * s, 16)] = zero
        return 0
    lax.fori_loop(0, CHUNK, zb, 0)

    def zs(k, _):
        pltpu.sync_copy(buf_a.at[pl.ds(0, ZCH)],
                        acc_sh.at[pl.ds(t * RPT + k * ZCH, ZCH)])
        return 0
    lax.fori_loop(0, ZN, zs, 0)
    pltpu.sync_copy(buf_a.at[pl.ds(0, ZTAIL)],
                    acc_sh.at[pl.ds(t * RPT + ZN * ZCH, ZTAIL)])
    plsc.subcore_barrier()

    pv = pp_v[0, pl.ds(0, 16)]
    t_sc = pv[0]
    teps = pv[1]
    cf = jnp.full((L,), c, jnp.int32).astype(jnp.float32)
    s0v = 1.0 - cf
    s1v = cf

    def start_side(k, src_ref, rows_ref, e_ref, semg, seme):
        ck = t * CPT + k
        pltpu.sync_copy(src_hbm.at[ck], src_ref)
        pltpu.async_copy(g_hbm.at[src_ref], rows_ref, semg)
        pltpu.async_copy(e_hbm.at[pl.ds(ck * CHUNK, CHUNK)], e_ref, seme)

    def wait_side(src_ref, rows_ref, e_ref, semg, seme):
        pltpu.make_async_copy(g_hbm.at[src_ref], rows_ref, semg).wait()
        pltpu.make_async_copy(e_hbm.at[pl.ds(0, CHUNK)], e_ref, seme).wait()

    def start_scatter(buf_ref, dst_ref, semx):
        pltpu.async_copy(buf_ref, acc_sh.at[dst_ref], semx, add=True)

    def wait_scatter(buf_ref, dst_ref, semx):
        pltpu.make_async_copy(buf_ref, acc_sh.at[dst_ref], semx).wait()

    def compute_chunk(k, rows_ref, e_ref, buf_ref, dst_ref):
        pltpu.sync_copy(dst_hbm.at[t * CPT + k], dst_ref)

        def edge_body(j):
            for s in range(8):
                sl = pl.ds(16 * s, 16)
                u = jnp.maximum(t_sc * rows_ref[j, sl] + e_ref[j, sl],
                                0.0) + teps
                w = jnp.exp(u)
                buf_ref[j, sl] = w * (s0v + s1v * u)
        plsc.parallel_loop(0, CHUNK, 1, unroll=4)(edge_body)

    def arm(j, _):
        trash = jnp.full((L,), N_NODES, jnp.int32)
        dst_a[pl.ds(j * 16, 16)] = trash
        dst_b[pl.ds(j * 16, 16)] = trash
        return 0
    lax.fori_loop(0, CHUNK // 16, arm, 0)
    start_scatter(buf_a, dst_a, sem_sa)
    start_scatter(buf_b, dst_b, sem_sb)
    start_side(0, src_a, rows_a, e_a, sem_ga, sem_ea)

    def pair_body(i, _):
        k0 = 2 * i
        wait_side(src_a, rows_a, e_a, sem_ga, sem_ea)
        start_side(k0 + 1, src_b, rows_b, e_b, sem_gb, sem_eb)
        wait_scatter(buf_a, dst_a, sem_sa)
        compute_chunk(k0, rows_a, e_a, buf_a, dst_a)
        start_scatter(buf_a, dst_a, sem_sa)
        wait_side(src_b, rows_b, e_b, sem_gb, sem_eb)

        @pl.when(i < CPT // 2 - 1)
        def _():
            start_side(k0 + 2, src_a, rows_a, e_a, sem_ga, sem_ea)

        wait_scatter(buf_b, dst_b, sem_sb)
        compute_chunk(k0 + 1, rows_b, e_b, buf_b, dst_b)
        start_scatter(buf_b, dst_b, sem_sb)
        return 0
    lax.fori_loop(0, CPT // 2, pair_body, 0)
    wait_scatter(buf_a, dst_a, sem_sa)
    wait_scatter(buf_b, dst_b, sem_sb)
    plsc.subcore_barrier()

    def wb(k, _):
        r = t * RPT + k * ZCH
        pltpu.sync_copy(acc_sh.at[pl.ds(r, ZCH)], buf_a.at[pl.ds(0, ZCH)])
        pltpu.sync_copy(buf_a.at[pl.ds(0, ZCH)],
                        out_hbm.at[pl.ds(c * NROWS + r, ZCH)])
        return 0
    lax.fori_loop(0, ZN, wb, 0)
    rt = t * RPT + ZN * ZCH
    pltpu.sync_copy(acc_sh.at[pl.ds(rt, ZTAIL)], buf_a.at[pl.ds(0, ZTAIL)])
    pltpu.sync_copy(buf_a.at[pl.ds(0, ZTAIL)],
                    out_hbm.at[pl.ds(c * NROWS + rt, ZTAIL)])


def _sc_edge(g, src2, dst2, e2, pp):
    f32 = jnp.float32
    mesh = plsc.VectorSubcoreMesh(core_axis_name="c", subcore_axis_name="s",
                                  num_cores=NC, num_subcores=NS)
    kern = pl.kernel(
        _sc_edge_body,
        out_type=jax.ShapeDtypeStruct((2 * NROWS, D), f32),
        mesh=mesh,
        scratch_types=[
            pltpu.VMEM((CHUNK,), jnp.int32),
            pltpu.VMEM((CHUNK,), jnp.int32),
            pltpu.VMEM((CHUNK,), jnp.int32),
            pltpu.VMEM((CHUNK,), jnp.int32),
            pltpu.VMEM((CHUNK, D), f32),
            pltpu.VMEM((CHUNK, D), f32),
            pltpu.VMEM((CHUNK, D), f32),
            pltpu.VMEM((CHUNK, D), f32),
            pltpu.VMEM((CHUNK, D), f32),
            pltpu.VMEM((CHUNK, D), f32),
            pltpu.VMEM((1, D), f32),
            pltpu.SemaphoreType.DMA,
            pltpu.SemaphoreType.DMA,
            pltpu.SemaphoreType.DMA,
            pltpu.SemaphoreType.DMA,
            pltpu.SemaphoreType.DMA,
            pltpu.SemaphoreType.DMA,
            pltpu.VMEM_SHARED((NROWS, D), f32),
        ],
    )
    res = kern(g, src2, dst2, e2, pp)
    return res[:N_NODES], res[NROWS:NROWS + N_NODES]


def _edge_phase(g, src2, dst2, e2, t):
    pp = jnp.concatenate([jnp.full((1, 1), t, jnp.float32),
                          jnp.full((1, 1), t * EPS_MSG, jnp.float32),
                          jnp.zeros((1, D - 2), jnp.float32)], axis=1)
    return _sc_edge(g, src2, dst2, e2, pp)


def kernel(x, edge_index, edge_attr, batch, clinical, params):
    del clinical
    src, dst = edge_index[0], edge_index[1]
    n_edges = src.shape[0]
    npad = E_PAD - n_edges
    src2 = jnp.concatenate([src.astype(jnp.int32),
                            jnp.zeros((npad,), jnp.int32)]).reshape(NS * CPT, CHUNK)
    dst2 = jnp.concatenate([dst.astype(jnp.int32),
                            jnp.full((npad,), N_NODES, jnp.int32)]).reshape(NS * CPT, CHUNK)
    ea2d = jnp.concatenate([edge_attr.astype(jnp.float32),
                            jnp.zeros((npad, 4), jnp.float32)])
    bns = 1.0 / np.sqrt(1.0 + BN_EPS)

    def folded(i):
        p = params[f"conv{i}"]
        s = p["bn1_w"] * bns
        w1 = p["W1"] * s[None, :]
        b1 = (p["b1"] * s + p["bn1_b"])[None, :]
        w2 = p["W2"]
        b2 = p["b2"][None, :]
        return w1, b1, w2, b2

    def norm(name):
        nm = params[name]
        return (nm["w"] * bns)[None, :], nm["b"][None, :]

    nw1, nb1 = norm("norm1")
    nw2, nb2 = norm("norm2")
    nw0, nb0 = norm("norm0")
    batch3d = batch.astype(jnp.int32).reshape(N_NODES // BLK, 1, BLK)

    tinvs = [jnp.full((1, D), 1.0, jnp.float32) / params[f"conv{i}"]["t"]
             for i in range(3)]
    eprimes = []
    for i in range(3):
        p = params[f"conv{i}"]
        eprimes.append(_tc_edge_feat(ea2d, p["t"] * p["We"],
                                     (p["t"] * p["be"])[None, :]))

    s1, s2 = _edge_phase(x, src2, dst2, eprimes[0], params["conv0"]["t"])
    x1, g1 = _tc_layer(s1, s2, tinvs[0], x, x, *folded(0), nw1, nb1,
                       has_res=False)
    s1, s2 = _edge_phase(g1, src2, dst2, eprimes[1], params["conv1"]["t"])
    x2, g2 = _tc_layer(s1, s2, tinvs[1], g1, x1, *folded(1), nw2, nb2,
                       has_res=True)
    s1, s2 = _edge_phase(g2, src2, dst2, eprimes[2], params["conv2"]["t"])
    return _tc_final(s1, s2, tinvs[2], g2, x2, *folded(2), nw0, nb0, batch3d)
